# baseline (speedup 1.0000x reference)
import jax
import jax.numpy as jnp
from jax import lax
from jax.experimental import pallas as pl
from jax.experimental.pallas import tpu as pltpu
from jax.experimental.pallas import tpu_sc as plsc

B, F, H, D = 4096, 26, 20, 32
NC, NS = 2, 16
NW = NC * NS
B_W = B // NW
G = 16
ROWS_G = G * F
GROUPS_W = B_W // G
IDX_ROWS_W = GROUPS_W * H


def _body(idx_hbm, table_hbm, out_hbm, idx_v, acc0, acc1,
          gsem0, gsem1, osem0, osem1):
    wid = lax.axis_index("s") * NC + lax.axis_index("c")
    gbase = wid * GROUPS_W
    accs = (acc0, acc1)
    gsems = (gsem0, gsem1)
    osems = (osem0, osem1)

    pltpu.sync_copy(idx_hbm.at[pl.ds(wid * IDX_ROWS_W, IDX_ROWS_W)], idx_v)

    def zero(b):
        @plsc.parallel_loop(0, ROWS_G, 1, unroll=13)
        def _row(i):
            z = jnp.zeros((16,), jnp.float32)
            accs[b][0, i, pl.ds(0, 16)] = z
            accs[b][0, i, pl.ds(16, 16)] = z

    def issue(g, b):
        for j in range(H):
            pltpu.async_copy(
                table_hbm.at[idx_v.at[g * H + j]],
                accs[b].at[0],
                gsems[b], add=True)

    def wait_gather(b):
        for _ in range(H):
            pltpu.make_async_copy(
                table_hbm.at[pl.ds(0, ROWS_G)], accs[b].at[0],
                gsems[b]).wait()

    def wait_out(b):
        pltpu.make_async_copy(
            accs[b], out_hbm.at[pl.ds(gbase, 1)], osems[b]).wait()

    zero(0)
    issue(0, 0)

    def step(p, carry):
        for b in range(2):
            g = 2 * p + b
            nb = 1 - b

            @pl.when(g + 1 < GROUPS_W)
            def _():
                @pl.when(g >= 1)
                def _():
                    wait_out(nb)
                zero(nb)
                issue(g + 1, nb)

            wait_gather(b)
            pltpu.async_copy(
                accs[b], out_hbm.at[pl.ds(gbase + g, 1)], osems[b])
        return carry

    lax.fori_loop(0, GROUPS_W // 2, step, 0)
    wait_out(0)
    wait_out(1)


_kern = pl.kernel(
    _body,
    out_type=jax.ShapeDtypeStruct((B // G, ROWS_G, D), jnp.float32),
    mesh=plsc.VectorSubcoreMesh(core_axis_name="c", subcore_axis_name="s"),
    compiler_params=pltpu.CompilerParams(use_tc_tiling_on_sc=False),
    scratch_types=[
        pltpu.VMEM((IDX_ROWS_W, ROWS_G), jnp.int32),
        pltpu.VMEM((1, ROWS_G, D), jnp.float32),
        pltpu.VMEM((1, ROWS_G, D), jnp.float32),
        pltpu.SemaphoreType.DMA,
        pltpu.SemaphoreType.DMA,
        pltpu.SemaphoreType.DMA,
        pltpu.SemaphoreType.DMA,
    ],
)


@jax.jit
def kernel(x, emb_weight):
    idx = (x.astype(jnp.int32)
           .reshape(NW, GROUPS_W, G, F, H)
           .transpose(0, 1, 4, 2, 3)
           .reshape(NW * IDX_ROWS_W, ROWS_G))
    return _kern(idx, emb_weight).reshape(B, F, D)


# speedup vs baseline: 1.4181x; 1.4181x over previous
import jax
import jax.numpy as jnp
from jax import lax
from jax.experimental import pallas as pl
from jax.experimental.pallas import tpu as pltpu
from jax.experimental.pallas import tpu_sc as plsc

B, F, H, D = 4096, 26, 20, 32
NC, NS = 2, 16
NW = NC * NS
B_W = B // NW
G = 4
ROWS_G = G * F
GROUPS_W = B_W // G
IDX_ROWS_W = GROUPS_W * H


def _body(idx_hbm, table_hbm, out_hbm, idx_v, acc0, acc1,
          gsem0, gsem1, osem0, osem1):
    wid = lax.axis_index("s") * NC + lax.axis_index("c")
    gbase = wid * GROUPS_W
    accs = (acc0, acc1)
    gsems = (gsem0, gsem1)
    osems = (osem0, osem1)

    pltpu.sync_copy(idx_hbm.at[pl.ds(wid * IDX_ROWS_W, IDX_ROWS_W)], idx_v)

    def zero(b):
        @plsc.parallel_loop(0, ROWS_G, 1, unroll=13)
        def _row(i):
            z = jnp.zeros((16,), jnp.float32)
            accs[b][0, i, pl.ds(0, 16)] = z
            accs[b][0, i, pl.ds(16, 16)] = z

    def issue(g, b):
        for j in range(H):
            pltpu.async_copy(
                table_hbm.at[idx_v.at[g * H + j]],
                accs[b].at[0],
                gsems[b], add=True)

    def wait_gather(b):
        for _ in range(H):
            pltpu.make_async_copy(
                table_hbm.at[pl.ds(0, ROWS_G)], accs[b].at[0],
                gsems[b]).wait()

    def wait_out(b):
        pltpu.make_async_copy(
            accs[b], out_hbm.at[pl.ds(gbase, 1)], osems[b]).wait()

    zero(0)
    issue(0, 0)

    def step(p, carry):
        for b in range(2):
            g = 2 * p + b
            nb = 1 - b

            @pl.when(g + 1 < GROUPS_W)
            def _():
                @pl.when(g >= 1)
                def _():
                    wait_out(nb)
                zero(nb)
                issue(g + 1, nb)

            wait_gather(b)
            pltpu.async_copy(
                accs[b], out_hbm.at[pl.ds(gbase + g, 1)], osems[b])
        return carry

    lax.fori_loop(0, GROUPS_W // 2, step, 0)
    wait_out(0)
    wait_out(1)


_kern = pl.kernel(
    _body,
    out_type=jax.ShapeDtypeStruct((B // G, ROWS_G, D), jnp.float32),
    mesh=plsc.VectorSubcoreMesh(core_axis_name="c", subcore_axis_name="s"),
    compiler_params=pltpu.CompilerParams(use_tc_tiling_on_sc=False),
    scratch_types=[
        pltpu.VMEM((IDX_ROWS_W, ROWS_G), jnp.int32),
        pltpu.VMEM((1, ROWS_G, D), jnp.float32),
        pltpu.VMEM((1, ROWS_G, D), jnp.float32),
        pltpu.SemaphoreType.DMA,
        pltpu.SemaphoreType.DMA,
        pltpu.SemaphoreType.DMA,
        pltpu.SemaphoreType.DMA,
    ],
)


@jax.jit
def kernel(x, emb_weight):
    idx = (x.astype(jnp.int32)
           .reshape(NW, GROUPS_W, G, F, H)
           .transpose(0, 1, 4, 2, 3)
           .reshape(NW * IDX_ROWS_W, ROWS_G))
    return _kern(idx, emb_weight).reshape(B, F, D)


# speedup vs baseline: 1.4757x; 1.0406x over previous
import jax
import jax.numpy as jnp
from jax import lax
from jax.experimental import pallas as pl
from jax.experimental.pallas import tpu as pltpu
from jax.experimental.pallas import tpu_sc as plsc

B, F, H, D = 4096, 26, 20, 32
NC, NS = 2, 16
NW = NC * NS
B_W = B // NW
G = 8
ROWS_G = G * F
GROUPS_W = B_W // G
IDX_ROWS_W = GROUPS_W * H


def _body(idx_hbm, table_hbm, out_hbm, idx_v, acc0, acc1,
          gsem0, gsem1, osem0, osem1):
    wid = lax.axis_index("s") * NC + lax.axis_index("c")
    gbase = wid * GROUPS_W
    accs = (acc0, acc1)
    gsems = (gsem0, gsem1)
    osems = (osem0, osem1)

    pltpu.sync_copy(idx_hbm.at[pl.ds(wid * IDX_ROWS_W, IDX_ROWS_W)], idx_v)

    def zero(b):
        @plsc.parallel_loop(0, ROWS_G, 1, unroll=13)
        def _row(i):
            z = jnp.zeros((16,), jnp.float32)
            accs[b][0, i, pl.ds(0, 16)] = z
            accs[b][0, i, pl.ds(16, 16)] = z

    def issue(g, b):
        for j in range(H):
            pltpu.async_copy(
                table_hbm.at[idx_v.at[g * H + j]],
                accs[b].at[0],
                gsems[b], add=True)

    def wait_gather(b):
        for _ in range(H):
            pltpu.make_async_copy(
                table_hbm.at[pl.ds(0, ROWS_G)], accs[b].at[0],
                gsems[b]).wait()

    def wait_out(b):
        pltpu.make_async_copy(
            accs[b], out_hbm.at[pl.ds(gbase, 1)], osems[b]).wait()

    zero(0)
    issue(0, 0)

    def step(p, carry):
        for b in range(2):
            g = 2 * p + b
            nb = 1 - b

            @pl.when(g + 1 < GROUPS_W)
            def _():
                @pl.when(g >= 1)
                def _():
                    wait_out(nb)
                zero(nb)
                issue(g + 1, nb)

            wait_gather(b)
            pltpu.async_copy(
                accs[b], out_hbm.at[pl.ds(gbase + g, 1)], osems[b])
        return carry

    lax.fori_loop(0, GROUPS_W // 2, step, 0)
    wait_out(0)
    wait_out(1)


_kern = pl.kernel(
    _body,
    out_type=jax.ShapeDtypeStruct((B // G, ROWS_G, D), jnp.float32),
    mesh=plsc.VectorSubcoreMesh(core_axis_name="c", subcore_axis_name="s"),
    compiler_params=pltpu.CompilerParams(use_tc_tiling_on_sc=False),
    scratch_types=[
        pltpu.VMEM((IDX_ROWS_W, ROWS_G), jnp.int32),
        pltpu.VMEM((1, ROWS_G, D), jnp.float32),
        pltpu.VMEM((1, ROWS_G, D), jnp.float32),
        pltpu.SemaphoreType.DMA,
        pltpu.SemaphoreType.DMA,
        pltpu.SemaphoreType.DMA,
        pltpu.SemaphoreType.DMA,
    ],
)


@jax.jit
def kernel(x, emb_weight):
    idx = (x.astype(jnp.int32)
           .reshape(NW, GROUPS_W, G, F, H)
           .transpose(0, 1, 4, 2, 3)
           .reshape(NW * IDX_ROWS_W, ROWS_G))
    return _kern(idx, emb_weight).reshape(B, F, D)
